# Initial kernel scaffold; baseline (speedup 1.0000x reference)
import jax
import jax.numpy as jnp
from jax.experimental import pallas as pl


def kernel(x, edge_index, W1, b1, W2, b2):
    raise NotImplementedError("write your pallas kernel here")


# speedup vs baseline: 23.1223x; 23.1223x over previous
import functools

import jax
import jax.numpy as jnp
from jax import lax
from jax.experimental import pallas as pl
from jax.experimental.pallas import tpu as pltpu
from jax.experimental.pallas import tpu_sc as plsc

N = 100000
NP_ = 102400
E = 6400000
C = 128
ROWS = E // C
NSC = 2
NSUB = 16
NW = NSC * NSUB
ROWS_LO = ROWS // NW
ROWS_EXTRA = ROWS % NW
NPT = NP_ // NSUB

_mesh = plsc.VectorSubcoreMesh(core_axis_name="c", subcore_axis_name="s")


def _worker(c, s):
    wid = c * NSUB + s
    base = wid * ROWS_LO + jnp.minimum(wid, ROWS_EXTRA)
    nrows = jnp.where(wid < ROWS_EXTRA, ROWS_LO + 1, ROWS_LO)
    return base, nrows


@functools.partial(
    pl.kernel,
    out_type=jax.ShapeDtypeStruct((NSC, NP_), jnp.float32),
    mesh=_mesh,
    scratch_types=[
        pltpu.VMEM((C,), jnp.int32),
        pltpu.VMEM((C,), jnp.float32),
        pltpu.VMEM_SHARED((NP_,), jnp.float32),
    ],
)
def _deg_kernel(dst_hbm, zeros_hbm, out_hbm, idx_v, ones_v, acc_sh):
    c = lax.axis_index("c")
    s = lax.axis_index("s")
    pltpu.sync_copy(zeros_hbm, acc_sh.at[pl.ds(s * NPT, NPT)])
    for i in range(C // 16):
        ones_v[pl.ds(i * 16, 16)] = jnp.full((16,), 1.0, jnp.float32)
    plsc.subcore_barrier()

    base, nrows = _worker(c, s)

    def body(i, carry):
        pltpu.sync_copy(dst_hbm.at[base + i], idx_v)
        pltpu.sync_copy(ones_v, acc_sh.at[idx_v], add=True)
        return carry

    lax.fori_loop(0, nrows, body, 0)
    plsc.subcore_barrier()
    pltpu.sync_copy(acc_sh.at[pl.ds(s * NPT, NPT)],
                    out_hbm.at[c, pl.ds(s * NPT, NPT)])


def _make_agg_kernel(F):
    @functools.partial(
        pl.kernel,
        out_type=jax.ShapeDtypeStruct((NSC, NP_, F), jnp.float32),
        mesh=_mesh,
        scratch_types=[
            pltpu.VMEM((C,), jnp.int32),
            pltpu.VMEM((C,), jnp.int32),
            pltpu.VMEM((C, F), jnp.float32),
            pltpu.SemaphoreType.DMA,
            pltpu.VMEM_SHARED((NP_, F), jnp.float32),
        ],
        compiler_params=pltpu.CompilerParams(use_tc_tiling_on_sc=False),
    )
    def _agg_kernel(g_hbm, src_hbm, dst_hbm, zeros_hbm, out_hbm,
                    src_v, dst_v, rows_v, sem, acc_sh):
        c = lax.axis_index("c")
        s = lax.axis_index("s")
        pltpu.sync_copy(zeros_hbm, acc_sh.at[pl.ds(s * NPT, NPT), :])
        plsc.subcore_barrier()

        base, nrows = _worker(c, s)

        def body(i, carry):
            pltpu.sync_copy(src_hbm.at[base + i], src_v)
            pltpu.sync_copy(dst_hbm.at[base + i], dst_v)
            pltpu.async_copy(g_hbm.at[src_v], rows_v, sem).wait()
            pltpu.sync_copy(rows_v, acc_sh.at[dst_v], add=True)
            return carry

        lax.fori_loop(0, nrows, body, 0)
        plsc.subcore_barrier()
        pltpu.sync_copy(acc_sh.at[pl.ds(s * NPT, NPT), :],
                        out_hbm.at[c, pl.ds(s * NPT, NPT), :])

    return _agg_kernel


_agg16 = _make_agg_kernel(16)
_agg8 = _make_agg_kernel(8)


RB = 2048
GRID = NP_ // RB


def _tc1_body(degp0_ref, degp1_ref, x_ref, w1_ref, d_ref, g1_ref):
    deg = degp0_ref[...] + degp1_ref[...] + 1.0
    d = lax.rsqrt(deg)
    h = jnp.dot(x_ref[...], w1_ref[...], preferred_element_type=jnp.float32)
    d_ref[...] = d
    g1_ref[...] = h * d


def _tc2_body(aggp_ref, g1_ref, d_ref, w2_ref, b1_ref, g2_ref):
    agg = aggp_ref[0] + aggp_ref[1]
    d = d_ref[...]
    out1 = jnp.maximum(d * (agg + g1_ref[...]) + b1_ref[...], 0.0)
    h2 = jnp.dot(out1, w2_ref[...], preferred_element_type=jnp.float32)
    g2_ref[...] = h2 * d


def _tc3_body(aggp_ref, g2_ref, d_ref, b2_ref, out_ref):
    agg = aggp_ref[0] + aggp_ref[1]
    d = d_ref[...]
    out = jnp.maximum(d * (agg + g2_ref[...]) + b2_ref[...], 0.0)
    out_ref[...] = out[:, :4]


def _tc1(degp0, degp1, x, W1):
    return pl.pallas_call(
        _tc1_body,
        grid=(GRID,),
        in_specs=[
            pl.BlockSpec((RB, 1), lambda i: (i, 0)),
            pl.BlockSpec((RB, 1), lambda i: (i, 0)),
            pl.BlockSpec((RB, 5), lambda i: (i, 0)),
            pl.BlockSpec((5, 16), lambda i: (0, 0)),
        ],
        out_specs=[
            pl.BlockSpec((RB, 1), lambda i: (i, 0)),
            pl.BlockSpec((RB, 16), lambda i: (i, 0)),
        ],
        out_shape=[
            jax.ShapeDtypeStruct((NP_, 1), jnp.float32),
            jax.ShapeDtypeStruct((NP_, 16), jnp.float32),
        ],
    )(degp0, degp1, x, W1)


def _tc2(agg1p, g1, d, W2p, b1):
    return pl.pallas_call(
        _tc2_body,
        grid=(GRID,),
        in_specs=[
            pl.BlockSpec((NSC, RB, 16), lambda i: (0, i, 0)),
            pl.BlockSpec((RB, 16), lambda i: (i, 0)),
            pl.BlockSpec((RB, 1), lambda i: (i, 0)),
            pl.BlockSpec((16, 8), lambda i: (0, 0)),
            pl.BlockSpec((1, 16), lambda i: (0, 0)),
        ],
        out_specs=pl.BlockSpec((RB, 8), lambda i: (i, 0)),
        out_shape=jax.ShapeDtypeStruct((NP_, 8), jnp.float32),
    )(agg1p, g1, d, W2p, b1)


def _tc3(agg2p, g2, d, b2p):
    return pl.pallas_call(
        _tc3_body,
        grid=(GRID,),
        in_specs=[
            pl.BlockSpec((NSC, RB, 8), lambda i: (0, i, 0)),
            pl.BlockSpec((RB, 8), lambda i: (i, 0)),
            pl.BlockSpec((RB, 1), lambda i: (i, 0)),
            pl.BlockSpec((1, 8), lambda i: (0, 0)),
        ],
        out_specs=pl.BlockSpec((RB, 4), lambda i: (i, 0)),
        out_shape=jax.ShapeDtypeStruct((NP_, 4), jnp.float32),
    )(agg2p, g2, d, b2p)


@jax.jit
def kernel(x, edge_index, W1, b1, W2, b2):
    ei = edge_index.astype(jnp.int32)
    src2d = ei[0].reshape(ROWS, C)
    dst2d = ei[1].reshape(ROWS, C)

    xp = jnp.pad(x, ((0, NP_ - N), (0, 0)))
    degp = _deg_kernel(dst2d, jnp.zeros((NPT,), jnp.float32))
    d, g1 = _tc1(degp[0].reshape(NP_, 1), degp[1].reshape(NP_, 1), xp, W1)

    agg1p = _agg16(g1, src2d, dst2d, jnp.zeros((NPT, 16), jnp.float32))
    W2p = jnp.pad(W2, ((0, 0), (0, 4)))
    g2 = _tc2(agg1p, g1, d, W2p, b1.reshape(1, 16))

    agg2p = _agg8(g2, src2d, dst2d, jnp.zeros((NPT, 8), jnp.float32))
    out = _tc3(agg2p, g2, d, jnp.pad(b2, (0, 4)).reshape(1, 8))
    return out[:N]


# speedup vs baseline: 93.5989x; 4.0480x over previous
import functools

import jax
import jax.numpy as jnp
from jax import lax
from jax.experimental import pallas as pl
from jax.experimental.pallas import tpu as pltpu
from jax.experimental.pallas import tpu_sc as plsc

N = 100000
NP_ = 102400
E = 6400000
C = 128
NSC = 2
NSUB = 16
NW = NSC * NSUB
W_ROWS = 1568
E_PAD = NW * W_ROWS * C
ROWS = E_PAD // C
NPT = NP_ // NSUB

_mesh = plsc.VectorSubcoreMesh(core_axis_name="c", subcore_axis_name="s")
_sc_params = pltpu.CompilerParams(use_tc_tiling_on_sc=False)


DG = 16
DNB = W_ROWS // DG

@functools.partial(
    pl.kernel,
    out_type=jax.ShapeDtypeStruct((NSC, NP_), jnp.float32),
    mesh=_mesh,
    scratch_types=[
        pltpu.VMEM((DG, C), jnp.int32),
        pltpu.VMEM((DG, C), jnp.int32),
        pltpu.VMEM((C,), jnp.float32),
        pltpu.SemaphoreType.DMA,
        pltpu.SemaphoreType.DMA,
        pltpu.VMEM_SHARED((NP_,), jnp.float32),
    ],
    compiler_params=_sc_params,
)
def _deg_kernel(dst_hbm, zeros_hbm, out_hbm, idxA, idxB, ones_v, ssA, ssB,
                acc_sh):
    c = lax.axis_index("c")
    s = lax.axis_index("s")
    pltpu.sync_copy(zeros_hbm, acc_sh.at[pl.ds(s * NPT, NPT)])
    for i in range(C // 16):
        ones_v[pl.ds(i * 16, 16)] = jnp.full((16,), 1.0, jnp.float32)
    plsc.subcore_barrier()
    base = (c * NSUB + s) * W_ROWS

    def fire(idx, sem, r0):
        pltpu.sync_copy(dst_hbm.at[pl.ds(r0, DG)], idx)
        for j in range(DG):
            pltpu.async_copy(ones_v, acc_sh.at[idx.at[j]], sem, add=True)

    def drain(idx, sem):
        for j in range(DG):
            pltpu.make_async_copy(ones_v, acc_sh.at[idx.at[j]], sem).wait()

    fire(idxA, ssA, base)
    fire(idxB, ssB, base + DG)

    def body(bb, carry):
        drain(idxA, ssA)
        fire(idxA, ssA, base + (2 * bb) * DG)
        drain(idxB, ssB)
        fire(idxB, ssB, base + (2 * bb + 1) * DG)
        return carry

    lax.fori_loop(1, DNB // 2, body, 0)
    drain(idxA, ssA)
    drain(idxB, ssB)
    plsc.subcore_barrier()
    pltpu.sync_copy(acc_sh.at[pl.ds(s * NPT, NPT)],
                    out_hbm.at[c, pl.ds(s * NPT, NPT)])


def _make_agg_kernel(F, G):
    NB = W_ROWS // G
    @functools.partial(
        pl.kernel,
        out_type=jax.ShapeDtypeStruct((NSC, NP_, F), jnp.float32),
        mesh=_mesh,
        scratch_types=[
            pltpu.VMEM((G, C), jnp.int32),
            pltpu.VMEM((G, C), jnp.int32),
            pltpu.VMEM((G, C), jnp.int32),
            pltpu.VMEM((G, C), jnp.int32),
            pltpu.VMEM((G, C, F), jnp.float32),
            pltpu.VMEM((G, C, F), jnp.float32),
            pltpu.SemaphoreType.DMA,
            pltpu.SemaphoreType.DMA,
            pltpu.SemaphoreType.DMA,
            pltpu.SemaphoreType.DMA,
            pltpu.VMEM_SHARED((NP_, F), jnp.float32),
        ],
        compiler_params=_sc_params,
    )
    def _agg_kernel(g_hbm, src_hbm, dst_hbm, zeros_hbm, out_hbm,
                    srcA, srcB, dstA, dstB, rowsA, rowsB,
                    gsA, gsB, ssA, ssB, acc_sh):
        c = lax.axis_index("c")
        s = lax.axis_index("s")
        pltpu.sync_copy(zeros_hbm, acc_sh.at[pl.ds(s * NPT, NPT), :])
        plsc.subcore_barrier()
        base = (c * NSUB + s) * W_ROWS

        def load_fire_gathers(srci, dsti, rows, gsem, r0):
            pltpu.sync_copy(src_hbm.at[pl.ds(r0, G)], srci)
            pltpu.sync_copy(dst_hbm.at[pl.ds(r0, G)], dsti)
            for j in range(G):
                pltpu.async_copy(g_hbm.at[srci.at[j]], rows.at[j], gsem)

        def drain_gathers(srci, rows, gsem):
            for j in range(G):
                pltpu.make_async_copy(g_hbm.at[srci.at[j]], rows.at[j],
                                      gsem).wait()

        def fire_scatters(dsti, rows, ssem):
            for j in range(G):
                pltpu.async_copy(rows.at[j], acc_sh.at[dsti.at[j]], ssem,
                                 add=True)

        def drain_scatters(dsti, rows, ssem):
            for j in range(G):
                pltpu.make_async_copy(rows.at[j], acc_sh.at[dsti.at[j]],
                                      ssem).wait()

        load_fire_gathers(srcA, dstA, rowsA, gsA, base)
        load_fire_gathers(srcB, dstB, rowsB, gsB, base + G)

        def body(bb, carry):
            rA = base + (2 * bb) * G
            drain_gathers(srcA, rowsA, gsA)
            fire_scatters(dstA, rowsA, ssA)
            drain_gathers(srcB, rowsB, gsB)
            fire_scatters(dstB, rowsB, ssB)
            drain_scatters(dstA, rowsA, ssA)
            load_fire_gathers(srcA, dstA, rowsA, gsA, rA + 2 * G)
            drain_scatters(dstB, rowsB, ssB)
            load_fire_gathers(srcB, dstB, rowsB, gsB, rA + 3 * G)
            return carry

        lax.fori_loop(0, NB // 2 - 1, body, 0)
        drain_gathers(srcA, rowsA, gsA)
        fire_scatters(dstA, rowsA, ssA)
        drain_gathers(srcB, rowsB, gsB)
        fire_scatters(dstB, rowsB, ssB)
        drain_scatters(dstA, rowsA, ssA)
        drain_scatters(dstB, rowsB, ssB)
        plsc.subcore_barrier()
        pltpu.sync_copy(acc_sh.at[pl.ds(s * NPT, NPT), :],
                        out_hbm.at[c, pl.ds(s * NPT, NPT), :])

    return _agg_kernel


_agg16 = _make_agg_kernel(16, 4)
_agg8 = _make_agg_kernel(8, 16)


RB = 2048
GRID = NP_ // RB


def _tc1_body(degp0_ref, degp1_ref, x_ref, w1_ref, d_ref, g1_ref):
    deg = degp0_ref[...] + degp1_ref[...] + 1.0
    d = lax.rsqrt(deg)
    h = jnp.dot(x_ref[...], w1_ref[...], preferred_element_type=jnp.float32)
    d_ref[...] = d
    g1_ref[...] = h * d


def _tc2_body(aggp_ref, g1_ref, d_ref, w2_ref, b1_ref, g2_ref):
    agg = aggp_ref[0] + aggp_ref[1]
    d = d_ref[...]
    out1 = jnp.maximum(d * (agg + g1_ref[...]) + b1_ref[...], 0.0)
    h2 = jnp.dot(out1, w2_ref[...], preferred_element_type=jnp.float32)
    g2_ref[...] = h2 * d


def _tc3_body(aggp_ref, g2_ref, d_ref, b2_ref, out_ref):
    agg = aggp_ref[0] + aggp_ref[1]
    d = d_ref[...]
    out = jnp.maximum(d * (agg + g2_ref[...]) + b2_ref[...], 0.0)
    out_ref[...] = out[:, :4]


def _tc1(degp0, degp1, x, W1):
    return pl.pallas_call(
        _tc1_body,
        grid=(GRID,),
        in_specs=[
            pl.BlockSpec((RB, 1), lambda i: (i, 0)),
            pl.BlockSpec((RB, 1), lambda i: (i, 0)),
            pl.BlockSpec((RB, 5), lambda i: (i, 0)),
            pl.BlockSpec((5, 16), lambda i: (0, 0)),
        ],
        out_specs=[
            pl.BlockSpec((RB, 1), lambda i: (i, 0)),
            pl.BlockSpec((RB, 16), lambda i: (i, 0)),
        ],
        out_shape=[
            jax.ShapeDtypeStruct((NP_, 1), jnp.float32),
            jax.ShapeDtypeStruct((NP_, 16), jnp.float32),
        ],
    )(degp0, degp1, x, W1)


def _tc2(agg1p, g1, d, W2p, b1):
    return pl.pallas_call(
        _tc2_body,
        grid=(GRID,),
        in_specs=[
            pl.BlockSpec((NSC, RB, 16), lambda i: (0, i, 0)),
            pl.BlockSpec((RB, 16), lambda i: (i, 0)),
            pl.BlockSpec((RB, 1), lambda i: (i, 0)),
            pl.BlockSpec((16, 8), lambda i: (0, 0)),
            pl.BlockSpec((1, 16), lambda i: (0, 0)),
        ],
        out_specs=pl.BlockSpec((RB, 8), lambda i: (i, 0)),
        out_shape=jax.ShapeDtypeStruct((NP_, 8), jnp.float32),
    )(agg1p, g1, d, W2p, b1)


def _tc3(agg2p, g2, d, b2p):
    return pl.pallas_call(
        _tc3_body,
        grid=(GRID,),
        in_specs=[
            pl.BlockSpec((NSC, RB, 8), lambda i: (0, i, 0)),
            pl.BlockSpec((RB, 8), lambda i: (i, 0)),
            pl.BlockSpec((RB, 1), lambda i: (i, 0)),
            pl.BlockSpec((1, 8), lambda i: (0, 0)),
        ],
        out_specs=pl.BlockSpec((RB, 4), lambda i: (i, 0)),
        out_shape=jax.ShapeDtypeStruct((NP_, 4), jnp.float32),
    )(agg2p, g2, d, b2p)


@jax.jit
def kernel(x, edge_index, W1, b1, W2, b2):
    ei = edge_index.astype(jnp.int32)
    pad_idx = N + (jnp.arange(E_PAD - E, dtype=jnp.int32) % (NP_ - N))
    src2d = jnp.concatenate([ei[0], pad_idx]).reshape(ROWS, C)
    dst2d = jnp.concatenate([ei[1], pad_idx]).reshape(ROWS, C)

    xp = jnp.pad(x, ((0, NP_ - N), (0, 0)))
    degp = _deg_kernel(dst2d, jnp.zeros((NPT,), jnp.float32))
    d, g1 = _tc1(degp[0].reshape(NP_, 1), degp[1].reshape(NP_, 1), xp, W1)

    agg1p = _agg16(g1, src2d, dst2d, jnp.zeros((NPT, 16), jnp.float32))
    W2p = jnp.pad(W2, ((0, 0), (0, 4)))
    g2 = _tc2(agg1p, g1, d, W2p, b1.reshape(1, 16))

    agg2p = _agg8(g2, src2d, dst2d, jnp.zeros((NPT, 8), jnp.float32))
    out = _tc3(agg2p, g2, d, jnp.pad(b2, (0, 4)).reshape(1, 8))
    return out[:N]


# speedup vs baseline: 100.2309x; 1.0709x over previous
import functools

import jax
import jax.numpy as jnp
from jax import lax
from jax.experimental import pallas as pl
from jax.experimental.pallas import tpu as pltpu
from jax.experimental.pallas import tpu_sc as plsc

N = 100000
NP_ = 102400
E = 6400000
C = 128
NSC = 2
NSUB = 16
NW = NSC * NSUB
W_ROWS = 1568
E_PAD = NW * W_ROWS * C
ROWS = E_PAD // C
NPT = NP_ // NSUB
SG = 16

_mesh = plsc.VectorSubcoreMesh(core_axis_name="c", subcore_axis_name="s")
_sc_params = pltpu.CompilerParams(use_tc_tiling_on_sc=False)


DG = 16
DNB = W_ROWS // DG


@functools.partial(
    pl.kernel,
    out_type=jax.ShapeDtypeStruct((NSC, NP_), jnp.float32),
    mesh=_mesh,
    scratch_types=[
        pltpu.VMEM((DG, C), jnp.int32),
        pltpu.VMEM((DG, C), jnp.int32),
        pltpu.VMEM((C,), jnp.float32),
        pltpu.SemaphoreType.DMA,
        pltpu.SemaphoreType.DMA,
        pltpu.VMEM_SHARED((NP_,), jnp.float32),
    ],
    compiler_params=_sc_params,
)
def _deg_kernel(esd_hbm, zeros_hbm, out_hbm, idxA, idxB, ones_v, ssA, ssB,
                acc_sh):
    c = lax.axis_index("c")
    s = lax.axis_index("s")
    pltpu.sync_copy(zeros_hbm, acc_sh.at[pl.ds(s * NPT, NPT)])
    for i in range(C // 16):
        ones_v[pl.ds(i * 16, 16)] = jnp.full((16,), 1.0, jnp.float32)
    plsc.subcore_barrier()
    base = (c * NSUB + s) * W_ROWS

    def fire(idx, sem, r0):
        pltpu.sync_copy(esd_hbm.at[pl.ds(r0, DG), 1], idx)
        for j in range(DG):
            pltpu.async_copy(ones_v, acc_sh.at[idx.at[j]], sem, add=True)

    def drain(idx, sem):
        for j in range(DG):
            pltpu.make_async_copy(ones_v, acc_sh.at[idx.at[j]], sem).wait()

    fire(idxA, ssA, base)
    fire(idxB, ssB, base + DG)

    def body(bb, carry):
        drain(idxA, ssA)
        fire(idxA, ssA, base + (2 * bb) * DG)
        drain(idxB, ssB)
        fire(idxB, ssB, base + (2 * bb + 1) * DG)
        return carry

    lax.fori_loop(1, DNB // 2, body, 0)
    drain(idxA, ssA)
    drain(idxB, ssB)
    plsc.subcore_barrier()
    pltpu.sync_copy(acc_sh.at[pl.ds(s * NPT, NPT)],
                    out_hbm.at[c, pl.ds(s * NPT, NPT)])


def _make_agg_kernel(F, G):
    KG = SG // G
    NSB = W_ROWS // SG

    @functools.partial(
        pl.kernel,
        out_type=jax.ShapeDtypeStruct((NSC, NP_, F), jnp.float32),
        mesh=_mesh,
        scratch_types=[
            pltpu.VMEM((SG, 2, C), jnp.int32),
            pltpu.VMEM((SG, 2, C), jnp.int32),
            pltpu.VMEM((G, C), jnp.int32),
            pltpu.VMEM((G, C), jnp.int32),
            pltpu.VMEM((G, C, F), jnp.float32),
            pltpu.VMEM((G, C, F), jnp.float32),
            pltpu.SemaphoreType.DMA,
            pltpu.SemaphoreType.DMA,
            pltpu.SemaphoreType.DMA,
            pltpu.SemaphoreType.DMA,
            pltpu.SemaphoreType.DMA,
            pltpu.VMEM_SHARED((NP_, F), jnp.float32),
        ],
        compiler_params=_sc_params,
    )
    def _agg_kernel(g_hbm, esd_hbm, zeros_hbm, out_hbm,
                    iA, iB, sidxP, sidxQ, rowsP, rowsQ,
                    isA, isB, ssP, ssQ, gs, acc_sh):
        c = lax.axis_index("c")
        s = lax.axis_index("s")
        pltpu.sync_copy(zeros_hbm, acc_sh.at[pl.ds(s * NPT, NPT), :])
        plsc.subcore_barrier()
        base = (c * NSUB + s) * W_ROWS
        sidx = (sidxP, sidxQ)
        rows = (rowsP, rowsQ)
        ssem = (ssP, ssQ)

        def idx_fire(ib, sem, sb):
            r0 = jnp.minimum(base + sb * SG, ROWS - SG)
            pltpu.async_copy(esd_hbm.at[pl.ds(r0, SG)], ib, sem)

        def idx_wait(ib, sem):
            pltpu.make_async_copy(esd_hbm.at[pl.ds(0, SG)], ib, sem).wait()

        def subgroup(ib, k, par, skip_ss_wait):
            if not skip_ss_wait:
                for j in range(G):
                    pltpu.make_async_copy(rows[par].at[j],
                                          acc_sh.at[sidx[par].at[j]],
                                          ssem[par]).wait()
            for j in range(G):
                pltpu.async_copy(g_hbm.at[ib.at[k * G + j, 0]],
                                 rows[par].at[j], gs)
            for j in range(G):
                pltpu.make_async_copy(g_hbm.at[ib.at[k * G + j, 0]],
                                      rows[par].at[j], gs).wait()
            for j in range(G):
                for m in range(C // 16):
                    sidx[par][j, pl.ds(16 * m, 16)] = (
                        ib[k * G + j, 1, pl.ds(16 * m, 16)])
            for j in range(G):
                pltpu.async_copy(rows[par].at[j], acc_sh.at[sidx[par].at[j]],
                                 ssem[par], add=True)

        def side(ib, isem, sb, sb_static_par, first_pair):
            idx_wait(ib, isem)
            for k in range(KG):
                t_par = (sb_static_par * KG + k) % 2
                skip = first_pair and (sb_static_par * KG + k) < 2
                subgroup(ib, k, t_par, skip)

        idx_fire(iA, isA, 0)
        idx_fire(iB, isB, 1)
        side(iA, isA, 0, 0, True)
        idx_fire(iA, isA, 2)
        side(iB, isB, 1, 1, True)
        idx_fire(iB, isB, 3)

        def body(bb, carry):
            side(iA, isA, 2 * bb, 0, False)
            idx_fire(iA, isA, 2 * bb + 2)
            side(iB, isB, 2 * bb + 1, 1, False)
            idx_fire(iB, isB, 2 * bb + 3)
            return carry

        lax.fori_loop(1, NSB // 2, body, 0)
        for par in (0, 1):
            for j in range(G):
                pltpu.make_async_copy(rows[par].at[j],
                                      acc_sh.at[sidx[par].at[j]],
                                      ssem[par]).wait()
        idx_wait(iA, isA)
        idx_wait(iB, isB)
        plsc.subcore_barrier()
        pltpu.sync_copy(acc_sh.at[pl.ds(s * NPT, NPT), :],
                        out_hbm.at[c, pl.ds(s * NPT, NPT), :])

    return _agg_kernel


_agg16 = _make_agg_kernel(16, 4)
_agg8 = _make_agg_kernel(8, 16)


RB = 4096
GRID = NP_ // RB


def _tc1_body(degp0_ref, degp1_ref, x_ref, w1_ref, d_ref, g1_ref):
    deg = degp0_ref[...] + degp1_ref[...] + 1.0
    d = lax.rsqrt(deg)
    h = jnp.dot(x_ref[...], w1_ref[...], preferred_element_type=jnp.float32)
    d_ref[...] = d
    g1_ref[...] = h * d


def _tc2_body(aggp_ref, g1_ref, d_ref, w2_ref, b1_ref, g2_ref):
    agg = aggp_ref[0] + aggp_ref[1]
    d = d_ref[...]
    out1 = jnp.maximum(d * (agg + g1_ref[...]) + b1_ref[...], 0.0)
    h2 = jnp.dot(out1, w2_ref[...], preferred_element_type=jnp.float32)
    g2_ref[...] = h2 * d


def _tc3_body(aggp_ref, g2_ref, d_ref, b2_ref, out_ref):
    agg = aggp_ref[0] + aggp_ref[1]
    d = d_ref[...]
    out = jnp.maximum(d * (agg + g2_ref[...]) + b2_ref[...], 0.0)
    out_ref[...] = out[:, :4]


def _tc1(degp0, degp1, x, W1):
    return pl.pallas_call(
        _tc1_body,
        grid=(GRID,),
        in_specs=[
            pl.BlockSpec((RB, 1), lambda i: (i, 0)),
            pl.BlockSpec((RB, 1), lambda i: (i, 0)),
            pl.BlockSpec((RB, 5), lambda i: (i, 0)),
            pl.BlockSpec((5, 16), lambda i: (0, 0)),
        ],
        out_specs=[
            pl.BlockSpec((RB, 1), lambda i: (i, 0)),
            pl.BlockSpec((RB, 16), lambda i: (i, 0)),
        ],
        out_shape=[
            jax.ShapeDtypeStruct((NP_, 1), jnp.float32),
            jax.ShapeDtypeStruct((NP_, 16), jnp.float32),
        ],
    )(degp0, degp1, x, W1)


def _tc2(agg1p, g1, d, W2p, b1):
    return pl.pallas_call(
        _tc2_body,
        grid=(GRID,),
        in_specs=[
            pl.BlockSpec((NSC, RB, 16), lambda i: (0, i, 0)),
            pl.BlockSpec((RB, 16), lambda i: (i, 0)),
            pl.BlockSpec((RB, 1), lambda i: (i, 0)),
            pl.BlockSpec((16, 8), lambda i: (0, 0)),
            pl.BlockSpec((1, 16), lambda i: (0, 0)),
        ],
        out_specs=pl.BlockSpec((RB, 8), lambda i: (i, 0)),
        out_shape=jax.ShapeDtypeStruct((NP_, 8), jnp.float32),
    )(agg1p, g1, d, W2p, b1)


def _tc3(agg2p, g2, d, b2p):
    return pl.pallas_call(
        _tc3_body,
        grid=(GRID,),
        in_specs=[
            pl.BlockSpec((NSC, RB, 8), lambda i: (0, i, 0)),
            pl.BlockSpec((RB, 8), lambda i: (i, 0)),
            pl.BlockSpec((RB, 1), lambda i: (i, 0)),
            pl.BlockSpec((1, 8), lambda i: (0, 0)),
        ],
        out_specs=pl.BlockSpec((RB, 4), lambda i: (i, 0)),
        out_shape=jax.ShapeDtypeStruct((NP_, 4), jnp.float32),
    )(agg2p, g2, d, b2p)


@jax.jit
def kernel(x, edge_index, W1, b1, W2, b2):
    ei = edge_index.astype(jnp.int32)
    pad_idx = N + (jnp.arange(E_PAD - E, dtype=jnp.int32) % (NP_ - N))
    src2d = jnp.concatenate([ei[0], pad_idx]).reshape(ROWS, C)
    dst2d = jnp.concatenate([ei[1], pad_idx]).reshape(ROWS, C)
    esd = jnp.stack([src2d, dst2d], axis=1)

    xp = jnp.pad(x, ((0, NP_ - N), (0, 0)))
    degp = _deg_kernel(esd, jnp.zeros((NPT,), jnp.float32))
    d, g1 = _tc1(degp[0].reshape(NP_, 1), degp[1].reshape(NP_, 1), xp, W1)

    agg1p = _agg16(g1, esd, jnp.zeros((NPT, 16), jnp.float32))
    W2p = jnp.pad(W2, ((0, 0), (0, 4)))
    g2 = _tc2(agg1p, g1, d, W2p, b1.reshape(1, 16))

    agg2p = _agg8(g2, esd, jnp.zeros((NPT, 8), jnp.float32))
    out = _tc3(agg2p, g2, d, jnp.pad(b2, (0, 4)).reshape(1, 8))
    return out[:N]


# speedup vs baseline: 125.8300x; 1.2554x over previous
import functools

import jax
import jax.numpy as jnp
from jax import lax
from jax.experimental import pallas as pl
from jax.experimental.pallas import tpu as pltpu
from jax.experimental.pallas import tpu_sc as plsc

N = 100000
NP_ = 102400
E = 6400000
C = 128
NSC = 2
NSUB = 16
NW = NSC * NSUB
ROWS = E // C
NPT = NP_ // NSUB
SG = 16
NSB = 97
ROWS_LO = ROWS // NW
ROWS_EXTRA = ROWS % NW
F16 = 16
NF = NP_ // F16

_mesh = plsc.VectorSubcoreMesh(core_axis_name="c", subcore_axis_name="s")
_sc_params = pltpu.CompilerParams(use_tc_tiling_on_sc=False)


def _worker(c, s):
    wid = c * NSUB + s
    base = wid * ROWS_LO + jnp.minimum(wid, ROWS_EXTRA)
    nrows = jnp.where(wid < ROWS_EXTRA, ROWS_LO + 1, ROWS_LO)
    return base, nrows


@functools.partial(
    pl.kernel,
    out_type=jax.ShapeDtypeStruct((NSC, NP_), jnp.float32),
    mesh=_mesh,
    scratch_types=[
        pltpu.VMEM((SG, C), jnp.int32),
        pltpu.VMEM((SG, C), jnp.int32),
        pltpu.VMEM((C,), jnp.float32),
        pltpu.SemaphoreType.DMA,
        pltpu.SemaphoreType.DMA,
        pltpu.VMEM_SHARED((NP_,), jnp.float32),
    ],
    compiler_params=_sc_params,
)
def _deg_kernel(ei_hbm, zeros_hbm, out_hbm, idxA, idxB, ones_v, ssA, ssB,
                acc_sh):
    c = lax.axis_index("c")
    s = lax.axis_index("s")
    pltpu.sync_copy(zeros_hbm, acc_sh.at[pl.ds(s * NPT, NPT)])
    for i in range(C // 16):
        ones_v[pl.ds(i * 16, 16)] = jnp.full((16,), 1.0, jnp.float32)
    plsc.subcore_barrier()
    base, nrows = _worker(c, s)

    def fire(idx, sem, r0):
        pltpu.sync_copy(ei_hbm.at[1, pl.ds(r0, SG)], idx)
        for j in range(SG):
            pltpu.async_copy(ones_v, acc_sh.at[idx.at[j]], sem, add=True)

    def drain(idx, sem):
        for j in range(SG):
            pltpu.make_async_copy(ones_v, acc_sh.at[idx.at[j]], sem).wait()

    fire(idxA, ssA, base)
    fire(idxB, ssB, base + SG)

    def body(bb, carry):
        drain(idxA, ssA)
        fire(idxA, ssA, base + (2 * bb) * SG)
        drain(idxB, ssB)
        fire(idxB, ssB, base + (2 * bb + 1) * SG)
        return carry

    lax.fori_loop(1, NSB // 2, body, 0)
    drain(idxA, ssA)
    fire(idxA, ssA, base + (NSB - 1) * SG)
    drain(idxB, ssB)
    drain(idxA, ssA)

    def tail(r, carry):
        pltpu.sync_copy(ei_hbm.at[1, base + NSB * SG + r], idxA.at[0])
        pltpu.sync_copy(ones_v, acc_sh.at[idxA.at[0]], add=True)
        return carry

    lax.fori_loop(0, nrows - NSB * SG, tail, 0)
    plsc.subcore_barrier()
    pltpu.sync_copy(acc_sh.at[pl.ds(s * NPT, NPT)],
                    out_hbm.at[c, pl.ds(s * NPT, NPT)])


def _make_agg_kernel(F, G):
    KG = SG // G

    @functools.partial(
        pl.kernel,
        out_type=jax.ShapeDtypeStruct((NSC, NP_, F), jnp.float32),
        mesh=_mesh,
        scratch_types=[
            pltpu.VMEM((SG, 2, C), jnp.int32),
            pltpu.VMEM((SG, 2, C), jnp.int32),
            pltpu.VMEM((G, C), jnp.int32),
            pltpu.VMEM((G, C), jnp.int32),
            pltpu.VMEM((G, C, F), jnp.float32),
            pltpu.VMEM((G, C, F), jnp.float32),
            pltpu.SemaphoreType.DMA,
            pltpu.SemaphoreType.DMA,
            pltpu.SemaphoreType.DMA,
            pltpu.SemaphoreType.DMA,
            pltpu.SemaphoreType.DMA,
            pltpu.VMEM_SHARED((NP_, F), jnp.float32),
        ],
        compiler_params=_sc_params,
    )
    def _agg_kernel(g_hbm, ei_hbm, zeros_hbm, out_hbm,
                    iA, iB, sidxP, sidxQ, rowsP, rowsQ,
                    isA, isB, ssP, ssQ, gs, acc_sh):
        c = lax.axis_index("c")
        s = lax.axis_index("s")
        pltpu.sync_copy(zeros_hbm, acc_sh.at[pl.ds(s * NPT, NPT), :])
        plsc.subcore_barrier()
        base, nrows = _worker(c, s)
        sidx = (sidxP, sidxQ)
        rows = (rowsP, rowsQ)
        ssem = (ssP, ssQ)

        def idx_fire(ib, sem, sb):
            r0 = jnp.minimum(base + sb * SG, ROWS - SG)
            pltpu.async_copy(ei_hbm.at[0, pl.ds(r0, SG)], ib.at[:, 0], sem)
            pltpu.async_copy(ei_hbm.at[1, pl.ds(r0, SG)], ib.at[:, 1], sem)

        def idx_wait(ib, sem):
            pltpu.make_async_copy(ei_hbm.at[0, pl.ds(0, SG)], ib.at[:, 0],
                                  sem).wait()
            pltpu.make_async_copy(ei_hbm.at[1, pl.ds(0, SG)], ib.at[:, 1],
                                  sem).wait()

        def subgroup(ib, k, par, skip_ss_wait):
            if not skip_ss_wait:
                for j in range(G):
                    pltpu.make_async_copy(rows[par].at[j],
                                          acc_sh.at[sidx[par].at[j]],
                                          ssem[par]).wait()
            for j in range(G):
                pltpu.async_copy(g_hbm.at[ib.at[k * G + j, 0]],
                                 rows[par].at[j], gs)
            for j in range(G):
                pltpu.make_async_copy(g_hbm.at[ib.at[k * G + j, 0]],
                                      rows[par].at[j], gs).wait()
            for j in range(G):
                for m in range(C // 16):
                    sidx[par][j, pl.ds(16 * m, 16)] = (
                        ib[k * G + j, 1, pl.ds(16 * m, 16)])
            for j in range(G):
                pltpu.async_copy(rows[par].at[j], acc_sh.at[sidx[par].at[j]],
                                 ssem[par], add=True)

        def side(ib, isem, sb, sb_static_par, first_pair):
            idx_wait(ib, isem)
            for k in range(KG):
                t_par = (sb_static_par * KG + k) % 2
                skip = first_pair and (sb_static_par * KG + k) < 2
                subgroup(ib, k, t_par, skip)

        idx_fire(iA, isA, 0)
        idx_fire(iB, isB, 1)
        side(iA, isA, 0, 0, True)
        idx_fire(iA, isA, 2)
        side(iB, isB, 1, 1, True)
        idx_fire(iB, isB, 3)

        def body(bb, carry):
            side(iA, isA, 2 * bb, 0, False)
            idx_fire(iA, isA, 2 * bb + 2)
            side(iB, isB, 2 * bb + 1, 1, False)
            idx_fire(iB, isB, 2 * bb + 3)
            return carry

        lax.fori_loop(1, NSB // 2, body, 0)
        side(iA, isA, NSB - 1, 0, False)
        for par in (0, 1):
            for j in range(G):
                pltpu.make_async_copy(rows[par].at[j],
                                      acc_sh.at[sidx[par].at[j]],
                                      ssem[par]).wait()
        idx_wait(iB, isB)

        def tail(r, carry):
            rr = base + NSB * SG + r
            pltpu.sync_copy(ei_hbm.at[0, rr], sidxP.at[0])
            pltpu.sync_copy(ei_hbm.at[1, rr], sidxQ.at[0])
            pltpu.async_copy(g_hbm.at[sidxP.at[0]], rowsP.at[0], gs).wait()
            pltpu.sync_copy(rowsP.at[0], acc_sh.at[sidxQ.at[0]], add=True)
            return carry

        lax.fori_loop(0, nrows - NSB * SG, tail, 0)
        plsc.subcore_barrier()
        pltpu.sync_copy(acc_sh.at[pl.ds(s * NPT, NPT), :],
                        out_hbm.at[c, pl.ds(s * NPT, NPT), :])

    return _agg_kernel


_agg16 = _make_agg_kernel(16, 4)
_agg8 = _make_agg_kernel(8, 16)


RBF = 1280
GRIDF = NF // RBF


def _dexp(d, rep):
    rbf = d.shape[0]
    return jnp.broadcast_to(d[:, :, None], (rbf, F16, rep)).reshape(
        rbf, F16 * rep)


def _split_matmul(a, wh_ref, wl_ref):
    ah = a.astype(jnp.bfloat16)
    al = (a - ah.astype(jnp.float32)).astype(jnp.bfloat16)
    wh = wh_ref[...]
    wl = wl_ref[...]
    h = jnp.dot(ah, wh, preferred_element_type=jnp.float32)
    h = h + jnp.dot(ah, wl, preferred_element_type=jnp.float32)
    h = h + jnp.dot(al, wh, preferred_element_type=jnp.float32)
    return h


def _tc1_body(degp_ref, x_ref, w1h_ref, w1l_ref, d_ref, g1_ref):
    deg = degp_ref[0] + degp_ref[1] + 1.0
    d = lax.rsqrt(deg)
    d_ref[...] = d
    h = _split_matmul(x_ref[...], w1h_ref, w1l_ref)
    g1_ref[...] = h * _dexp(d, 16)


def _tc2_body(aggp_ref, g1_ref, d_ref, w2h_ref, w2l_ref, b1_ref, g2_ref):
    agg = aggp_ref[0] + aggp_ref[1]
    d = d_ref[...]
    out1 = jnp.maximum(_dexp(d, 16) * (agg + g1_ref[...]) + b1_ref[...], 0.0)
    h2 = _split_matmul(out1, w2h_ref, w2l_ref)
    g2_ref[...] = h2 * _dexp(d, 8)


def _tc3_body(aggp_ref, g2_ref, d_ref, b2_ref, o_ref):
    agg = aggp_ref[0] + aggp_ref[1]
    dexp = _dexp(d_ref[...], 8)
    o_ref[...] = jnp.maximum(dexp * (agg + g2_ref[...]) + b2_ref[...], 0.0)


def _tc1(degp, x16, W1h, W1l):
    return pl.pallas_call(
        _tc1_body,
        grid=(GRIDF,),
        in_specs=[
            pl.BlockSpec((NSC, RBF, 16), lambda i: (0, i, 0)),
            pl.BlockSpec((RBF, 80), lambda i: (i, 0)),
            pl.BlockSpec((80, 256), lambda i: (0, 0)),
            pl.BlockSpec((80, 256), lambda i: (0, 0)),
        ],
        out_specs=[
            pl.BlockSpec((RBF, 16), lambda i: (i, 0)),
            pl.BlockSpec((RBF, 256), lambda i: (i, 0)),
        ],
        out_shape=[
            jax.ShapeDtypeStruct((NF, 16), jnp.float32),
            jax.ShapeDtypeStruct((NF, 256), jnp.float32),
        ],
    )(degp, x16, W1h, W1l)


def _tc2(agg1p, g1, d16, W2h, W2l, b1t):
    return pl.pallas_call(
        _tc2_body,
        grid=(GRIDF,),
        in_specs=[
            pl.BlockSpec((NSC, RBF, 256), lambda i: (0, i, 0)),
            pl.BlockSpec((RBF, 256), lambda i: (i, 0)),
            pl.BlockSpec((RBF, 16), lambda i: (i, 0)),
            pl.BlockSpec((256, 128), lambda i: (0, 0)),
            pl.BlockSpec((256, 128), lambda i: (0, 0)),
            pl.BlockSpec((1, 256), lambda i: (0, 0)),
        ],
        out_specs=pl.BlockSpec((RBF, 128), lambda i: (i, 0)),
        out_shape=jax.ShapeDtypeStruct((NF, 128), jnp.float32),
    )(agg1p, g1, d16, W2h, W2l, b1t)


def _tc3(agg2p, g2, d16, b2t):
    return pl.pallas_call(
        _tc3_body,
        grid=(GRIDF,),
        in_specs=[
            pl.BlockSpec((NSC, RBF, 128), lambda i: (0, i, 0)),
            pl.BlockSpec((RBF, 128), lambda i: (i, 0)),
            pl.BlockSpec((RBF, 16), lambda i: (i, 0)),
            pl.BlockSpec((1, 128), lambda i: (0, 0)),
        ],
        out_specs=pl.BlockSpec((RBF, 128), lambda i: (i, 0)),
        out_shape=jax.ShapeDtypeStruct((NF, 128), jnp.float32),
    )(agg2p, g2, d16, b2t)


@jax.jit
def kernel(x, edge_index, W1, b1, W2, b2):
    ei = edge_index.astype(jnp.int32).reshape(2, ROWS, C)

    eye16 = jnp.eye(F16, dtype=jnp.float32)
    W1blk = jnp.kron(eye16, W1)
    W2blk = jnp.kron(eye16, jnp.pad(W2, ((0, 0), (0, 4))))
    W1h = W1blk.astype(jnp.bfloat16)
    W1l = (W1blk - W1h.astype(jnp.float32)).astype(jnp.bfloat16)
    W2h = W2blk.astype(jnp.bfloat16)
    W2l = (W2blk - W2h.astype(jnp.float32)).astype(jnp.bfloat16)
    b1t = jnp.tile(b1, F16).reshape(1, 256)
    b2t = jnp.tile(jnp.pad(b2, (0, 4)), F16).reshape(1, 128)

    x16 = jnp.pad(x, ((0, NP_ - N), (0, 0))).reshape(NF, 80)
    degp = _deg_kernel(ei, jnp.zeros((NPT,), jnp.float32))
    d16, g1 = _tc1(degp.reshape(NSC, NF, 16), x16, W1h, W1l)

    agg1p = _agg16(g1.reshape(NP_, 16), ei, jnp.zeros((NPT, 16), jnp.float32))
    g2 = _tc2(agg1p.reshape(NSC, NF, 256), g1, d16, W2h, W2l, b1t)

    agg2p = _agg8(g2.reshape(NP_, 8), ei, jnp.zeros((NPT, 8), jnp.float32))
    o = _tc3(agg2p.reshape(NSC, NF, 128), g2, d16, b2t)
    return o.reshape(NP_, 8)[:N, :4]


# speedup vs baseline: 134.2026x; 1.0665x over previous
import functools

import jax
import jax.numpy as jnp
from jax import lax
from jax.experimental import pallas as pl
from jax.experimental.pallas import tpu as pltpu
from jax.experimental.pallas import tpu_sc as plsc

N = 100000
NP_ = 102400
E = 6400000
C = 128
NSC = 2
NSUB = 16
NW = NSC * NSUB
ROWS = E // C
NPT = NP_ // NSUB
SG = 16
NSB = 97
ROWS_LO = ROWS // NW
ROWS_EXTRA = ROWS % NW
F16 = 16
NF = NP_ // F16

_mesh = plsc.VectorSubcoreMesh(core_axis_name="c", subcore_axis_name="s")
_sc_params = pltpu.CompilerParams(use_tc_tiling_on_sc=False)


def _worker(c, s):
    wid = c * NSUB + s
    base = wid * ROWS_LO + jnp.minimum(wid, ROWS_EXTRA)
    nrows = jnp.where(wid < ROWS_EXTRA, ROWS_LO + 1, ROWS_LO)
    return base, nrows


@functools.partial(
    pl.kernel,
    out_type=jax.ShapeDtypeStruct((NSC, NP_), jnp.float32),
    mesh=_mesh,
    scratch_types=[
        pltpu.VMEM((SG, C), jnp.int32),
        pltpu.VMEM((SG, C), jnp.int32),
        pltpu.VMEM((C,), jnp.float32),
        pltpu.SemaphoreType.DMA,
        pltpu.SemaphoreType.DMA,
        pltpu.VMEM_SHARED((NP_,), jnp.float32),
    ],
    compiler_params=_sc_params,
)
def _deg_kernel(ei_hbm, zeros_hbm, out_hbm, idxA, idxB, ones_v, ssA, ssB,
                acc_sh):
    c = lax.axis_index("c")
    s = lax.axis_index("s")
    pltpu.sync_copy(zeros_hbm, acc_sh.at[pl.ds(s * NPT, NPT)])
    for i in range(C // 16):
        ones_v[pl.ds(i * 16, 16)] = jnp.full((16,), 1.0, jnp.float32)
    plsc.subcore_barrier()
    base, nrows = _worker(c, s)

    def fire(idx, sem, r0):
        pltpu.sync_copy(ei_hbm.at[1, pl.ds(r0, SG)], idx)
        for j in range(SG):
            pltpu.async_copy(ones_v, acc_sh.at[idx.at[j]], sem, add=True)

    def drain(idx, sem):
        for j in range(SG):
            pltpu.make_async_copy(ones_v, acc_sh.at[idx.at[j]], sem).wait()

    fire(idxA, ssA, base)
    fire(idxB, ssB, base + SG)

    def body(bb, carry):
        drain(idxA, ssA)
        fire(idxA, ssA, base + (2 * bb) * SG)
        drain(idxB, ssB)
        fire(idxB, ssB, base + (2 * bb + 1) * SG)
        return carry

    lax.fori_loop(1, NSB // 2, body, 0)
    drain(idxA, ssA)
    fire(idxA, ssA, base + (NSB - 1) * SG)
    drain(idxB, ssB)
    drain(idxA, ssA)

    def tail(r, carry):
        pltpu.sync_copy(ei_hbm.at[1, base + NSB * SG + r], idxA.at[0])
        pltpu.sync_copy(ones_v, acc_sh.at[idxA.at[0]], add=True)
        return carry

    lax.fori_loop(0, nrows - NSB * SG, tail, 0)
    plsc.subcore_barrier()
    pltpu.sync_copy(acc_sh.at[pl.ds(s * NPT, NPT)],
                    out_hbm.at[c, pl.ds(s * NPT, NPT)])


def _make_agg_kernel(F, G):
    KG = SG // G

    @functools.partial(
        pl.kernel,
        out_type=jax.ShapeDtypeStruct((NSC, NP_, F), jnp.float32),
        mesh=_mesh,
        scratch_types=[
            pltpu.VMEM((SG, 2, C), jnp.int32),
            pltpu.VMEM((SG, 2, C), jnp.int32),
            pltpu.VMEM((G, C), jnp.int32),
            pltpu.VMEM((G, C), jnp.int32),
            pltpu.VMEM((G, C, F), jnp.float32),
            pltpu.VMEM((G, C, F), jnp.float32),
            pltpu.SemaphoreType.DMA,
            pltpu.SemaphoreType.DMA,
            pltpu.SemaphoreType.DMA,
            pltpu.SemaphoreType.DMA,
            pltpu.SemaphoreType.DMA,
            pltpu.VMEM_SHARED((NP_, F), jnp.float32),
        ],
        compiler_params=_sc_params,
    )
    def _agg_kernel(g_hbm, ei_hbm, zeros_hbm, out_hbm,
                    iA, iB, sidxP, sidxQ, rowsP, rowsQ,
                    isA, isB, ssP, ssQ, gs, acc_sh):
        c = lax.axis_index("c")
        s = lax.axis_index("s")
        pltpu.sync_copy(zeros_hbm, acc_sh.at[pl.ds(s * NPT, NPT), :])
        plsc.subcore_barrier()
        base, nrows = _worker(c, s)
        sidx = (sidxP, sidxQ)
        rows = (rowsP, rowsQ)
        ssem = (ssP, ssQ)

        def idx_fire(ib, sem, sb):
            r0 = jnp.minimum(base + sb * SG, ROWS - SG)
            pltpu.async_copy(ei_hbm.at[0, pl.ds(r0, SG)], ib.at[:, 0], sem)
            pltpu.async_copy(ei_hbm.at[1, pl.ds(r0, SG)], ib.at[:, 1], sem)

        def idx_wait(ib, sem):
            pltpu.make_async_copy(ei_hbm.at[0, pl.ds(0, SG)], ib.at[:, 0],
                                  sem).wait()
            pltpu.make_async_copy(ei_hbm.at[1, pl.ds(0, SG)], ib.at[:, 1],
                                  sem).wait()

        def subgroup(ib, k, par, skip_ss_wait):
            if not skip_ss_wait:
                for j in range(G):
                    pltpu.make_async_copy(rows[par].at[j],
                                          acc_sh.at[sidx[par].at[j]],
                                          ssem[par]).wait()
            for j in range(G):
                pltpu.async_copy(g_hbm.at[ib.at[k * G + j, 0]],
                                 rows[par].at[j], gs)
            for j in range(G):
                pltpu.make_async_copy(g_hbm.at[ib.at[k * G + j, 0]],
                                      rows[par].at[j], gs).wait()
            for j in range(G):
                for m in range(C // 16):
                    sidx[par][j, pl.ds(16 * m, 16)] = (
                        ib[k * G + j, 1, pl.ds(16 * m, 16)])
            for j in range(G):
                pltpu.async_copy(rows[par].at[j], acc_sh.at[sidx[par].at[j]],
                                 ssem[par], add=True)

        def side(ib, isem, sb, sb_static_par, first_pair):
            idx_wait(ib, isem)
            for k in range(KG):
                t_par = (sb_static_par * KG + k) % 2
                skip = first_pair and (sb_static_par * KG + k) < 2
                subgroup(ib, k, t_par, skip)

        idx_fire(iA, isA, 0)
        idx_fire(iB, isB, 1)
        side(iA, isA, 0, 0, True)
        idx_fire(iA, isA, 2)
        side(iB, isB, 1, 1, True)
        idx_fire(iB, isB, 3)

        def body(bb, carry):
            side(iA, isA, 2 * bb, 0, False)
            idx_fire(iA, isA, 2 * bb + 2)
            side(iB, isB, 2 * bb + 1, 1, False)
            idx_fire(iB, isB, 2 * bb + 3)
            return carry

        lax.fori_loop(1, NSB // 2, body, 0)
        side(iA, isA, NSB - 1, 0, False)
        for par in (0, 1):
            for j in range(G):
                pltpu.make_async_copy(rows[par].at[j],
                                      acc_sh.at[sidx[par].at[j]],
                                      ssem[par]).wait()
        idx_wait(iB, isB)

        def tail(r, carry):
            rr = base + NSB * SG + r
            pltpu.sync_copy(ei_hbm.at[0, rr], sidxP.at[0])
            pltpu.sync_copy(ei_hbm.at[1, rr], sidxQ.at[0])
            pltpu.async_copy(g_hbm.at[sidxP.at[0]], rowsP.at[0], gs).wait()
            pltpu.sync_copy(rowsP.at[0], acc_sh.at[sidxQ.at[0]], add=True)
            return carry

        lax.fori_loop(0, nrows - NSB * SG, tail, 0)
        plsc.subcore_barrier()
        pltpu.sync_copy(acc_sh.at[pl.ds(s * NPT, NPT), :],
                        out_hbm.at[c, pl.ds(s * NPT, NPT), :])

    return _agg_kernel


_agg16 = _make_agg_kernel(16, 4)
_agg8 = _make_agg_kernel(8, 16)


RBF = 1280
GRIDF = NF // RBF


def _dexp(d, s_ref):
    dh = d.astype(jnp.bfloat16)
    dl = (d - dh.astype(jnp.float32)).astype(jnp.bfloat16)
    s = s_ref[...]
    return (jnp.dot(dh, s, preferred_element_type=jnp.float32) +
            jnp.dot(dl, s, preferred_element_type=jnp.float32))


def _split_matmul(a, wh_ref, wl_ref):
    ah = a.astype(jnp.bfloat16)
    al = (a - ah.astype(jnp.float32)).astype(jnp.bfloat16)
    wh = wh_ref[...]
    wl = wl_ref[...]
    h = jnp.dot(ah, wh, preferred_element_type=jnp.float32)
    h = h + jnp.dot(ah, wl, preferred_element_type=jnp.float32)
    h = h + jnp.dot(al, wh, preferred_element_type=jnp.float32)
    return h


def _tc1_body(degp_ref, x_ref, w1h_ref, w1l_ref, s16_ref, d_ref, g1_ref):
    deg = degp_ref[0] + degp_ref[1] + 1.0
    d0 = lax.rsqrt(deg)
    d = d0 * (1.5 - 0.5 * deg * d0 * d0)
    d_ref[...] = d
    h = _split_matmul(x_ref[...], w1h_ref, w1l_ref)
    g1_ref[...] = h * _dexp(d, s16_ref)


def _tc2_body(aggp_ref, g1_ref, d_ref, w2h_ref, w2l_ref, s16_ref, s16b_ref,
              b1_ref, g2_ref):
    agg = aggp_ref[0] + aggp_ref[1]
    d = d_ref[...]
    out1 = jnp.maximum(_dexp(d, s16_ref) * (agg + g1_ref[...]) + b1_ref[...],
                       0.0)
    h2 = _split_matmul(out1, w2h_ref, w2l_ref)
    g2_ref[...] = h2 * _dexp(d, s16b_ref)


def _tc3_body(aggp_ref, g2_ref, d_ref, s16b_ref, b2_ref, o_ref):
    agg = aggp_ref[0] + aggp_ref[1]
    dexp = _dexp(d_ref[...], s16b_ref)
    o_ref[...] = jnp.maximum(dexp * (agg + g2_ref[...]) + b2_ref[...], 0.0)


def _tc1(degp, x16, W1h, W1l, S16):
    return pl.pallas_call(
        _tc1_body,
        grid=(GRIDF,),
        in_specs=[
            pl.BlockSpec((NSC, RBF, 16), lambda i: (0, i, 0)),
            pl.BlockSpec((RBF, 80), lambda i: (i, 0)),
            pl.BlockSpec((80, 256), lambda i: (0, 0)),
            pl.BlockSpec((80, 256), lambda i: (0, 0)),
            pl.BlockSpec((16, 256), lambda i: (0, 0)),
        ],
        out_specs=[
            pl.BlockSpec((RBF, 16), lambda i: (i, 0)),
            pl.BlockSpec((RBF, 256), lambda i: (i, 0)),
        ],
        out_shape=[
            jax.ShapeDtypeStruct((NF, 16), jnp.float32),
            jax.ShapeDtypeStruct((NF, 256), jnp.float32),
        ],
    )(degp, x16, W1h, W1l, S16)


def _tc2(agg1p, g1, d16, W2h, W2l, S16, S16b, b1t):
    return pl.pallas_call(
        _tc2_body,
        grid=(GRIDF,),
        in_specs=[
            pl.BlockSpec((NSC, RBF, 256), lambda i: (0, i, 0)),
            pl.BlockSpec((RBF, 256), lambda i: (i, 0)),
            pl.BlockSpec((RBF, 16), lambda i: (i, 0)),
            pl.BlockSpec((256, 128), lambda i: (0, 0)),
            pl.BlockSpec((256, 128), lambda i: (0, 0)),
            pl.BlockSpec((16, 256), lambda i: (0, 0)),
            pl.BlockSpec((16, 128), lambda i: (0, 0)),
            pl.BlockSpec((1, 256), lambda i: (0, 0)),
        ],
        out_specs=pl.BlockSpec((RBF, 128), lambda i: (i, 0)),
        out_shape=jax.ShapeDtypeStruct((NF, 128), jnp.float32),
    )(agg1p, g1, d16, W2h, W2l, S16, S16b, b1t)


def _tc3(agg2p, g2, d16, S16b, b2t):
    return pl.pallas_call(
        _tc3_body,
        grid=(GRIDF,),
        in_specs=[
            pl.BlockSpec((NSC, RBF, 128), lambda i: (0, i, 0)),
            pl.BlockSpec((RBF, 128), lambda i: (i, 0)),
            pl.BlockSpec((RBF, 16), lambda i: (i, 0)),
            pl.BlockSpec((16, 128), lambda i: (0, 0)),
            pl.BlockSpec((1, 128), lambda i: (0, 0)),
        ],
        out_specs=pl.BlockSpec((RBF, 128), lambda i: (i, 0)),
        out_shape=jax.ShapeDtypeStruct((NF, 128), jnp.float32),
    )(agg2p, g2, d16, S16b, b2t)


@jax.jit
def kernel(x, edge_index, W1, b1, W2, b2):
    ei = edge_index.astype(jnp.int32).reshape(2, ROWS, C)

    eye16 = jnp.eye(F16, dtype=jnp.float32)
    W1blk = jnp.kron(eye16, W1)
    W2blk = jnp.kron(eye16, jnp.pad(W2, ((0, 0), (0, 4))))
    S16 = jnp.kron(eye16, jnp.ones((1, 16), jnp.float32)).astype(
        jnp.bfloat16)
    S16b = jnp.kron(eye16, jnp.ones((1, 8), jnp.float32)).astype(
        jnp.bfloat16)
    W1h = W1blk.astype(jnp.bfloat16)
    W1l = (W1blk - W1h.astype(jnp.float32)).astype(jnp.bfloat16)
    W2h = W2blk.astype(jnp.bfloat16)
    W2l = (W2blk - W2h.astype(jnp.float32)).astype(jnp.bfloat16)
    b1t = jnp.tile(b1, F16).reshape(1, 256)
    b2t = jnp.tile(jnp.pad(b2, (0, 4)), F16).reshape(1, 128)

    x16 = jnp.pad(x, ((0, NP_ - N), (0, 0))).reshape(NF, 80)
    degp = _deg_kernel(ei, jnp.zeros((NPT,), jnp.float32))
    d16, g1 = _tc1(degp.reshape(NSC, NF, 16), x16, W1h, W1l, S16)

    agg1p = _agg16(g1.reshape(NP_, 16), ei, jnp.zeros((NPT, 16), jnp.float32))
    g2 = _tc2(agg1p.reshape(NSC, NF, 256), g1, d16, W2h, W2l, S16, S16b, b1t)

    agg2p = _agg8(g2.reshape(NP_, 8), ei, jnp.zeros((NPT, 8), jnp.float32))
    o = _tc3(agg2p.reshape(NSC, NF, 128), g2, d16, S16b, b2t)
    return o.reshape(NP_, 8)[:N, :4]
